# baseline (speedup 1.0000x reference)
import functools

import jax
import jax.numpy as jnp
from jax import lax
from jax.experimental import pallas as pl
from jax.experimental.pallas import tpu as pltpu
from jax.experimental.pallas import tpu_sc as plsc

B = 4
C = 12
X = 64
V = X * X * X
XCH = 8
GX = X // XCH
RPG = XCH * X
RW = GX * RPG
LN = 2 * X
NW = 32
RPW = RW // NW


def _tc_stage1(pred_ref, targ_ref, lw_ref, out_ref):
    p = pred_ref[0]
    t = targ_ref[0]
    m = jnp.max(p, axis=0)
    s = jnp.sum(jnp.exp(p - m[None]), axis=0)
    lse = m + jnp.log(s)
    tp = jnp.sum(t * p, axis=0)
    ts = jnp.sum(t, axis=0)
    kl = (lse - tp).reshape(RPG, X)
    tsf = ts.reshape(RPG, X)
    lwf = lw_ref[0].reshape(RPG, X)
    valid = jnp.logical_and(lwf > 0, tsf == 1.0)
    out_ref[0, :, 0:X] = jnp.where(valid, kl, 0.0)
    out_ref[0, :, X:LN] = jnp.where(valid, 1.0, 0.0)


@functools.cache
def _build_stage1():
    return pl.pallas_call(
        _tc_stage1,
        grid=(B, GX),
        in_specs=[
            pl.BlockSpec((1, C, XCH, X, X), lambda b, g: (b, 0, g, 0, 0)),
            pl.BlockSpec((1, C, XCH, X, X), lambda b, g: (b, 0, g, 0, 0)),
            pl.BlockSpec((1, XCH, X, X), lambda b, g: (b, g, 0, 0)),
        ],
        out_specs=pl.BlockSpec((1, RPG, LN), lambda b, g: (b, g, 0)),
        out_shape=jax.ShapeDtypeStruct((B, RW, LN), jnp.float32),
    )


def _sc_body(rv_hbm, out_hbm, rv_v0, rv_v1, st_v, sems):
    w = lax.axis_index("s") * 2 + lax.axis_index("c")
    row0 = w * RPW
    bufs = (rv_v0, rv_v1)

    def start_tile(b, slot):
        pltpu.async_copy(rv_hbm.at[b, pl.ds(row0, RPW), :], bufs[slot],
                         sems.at[slot])

    def wait_tile(slot):
        pltpu.make_async_copy(rv_hbm.at[0, pl.ds(0, RPW), :], bufs[slot],
                              sems.at[slot]).wait()

    start_tile(0, 0)
    start_tile(1, 1)
    for b in range(B):
        slot = b % 2
        wait_tile(slot)
        rv = bufs[slot]

        def y_body(y, c2, rv=rv):
            s_acc, c_acc = c2
            for zg in range(4):
                s_acc = s_acc + rv[y, pl.ds(zg * 16, 16)]
            for zg in range(4, 8):
                c_acc = c_acc + rv[y, pl.ds(zg * 16, 16)]
            return s_acc, c_acc

        acc = lax.fori_loop(
            0, RPW, y_body,
            (jnp.zeros((16,), jnp.float32), jnp.zeros((16,), jnp.float32)))

        if b + 2 < B:
            start_tile(b + 2, slot)
        st_v[pl.ds(0, 16)] = acc[0]
        st_v[pl.ds(16, 16)] = acc[1]
        pltpu.sync_copy(st_v, out_hbm.at[b, w, :])


@functools.cache
def _build_sc_reduce():
    return pl.kernel(
        _sc_body,
        out_type=jax.ShapeDtypeStruct((B, NW, LN), jnp.float32),
        mesh=plsc.VectorSubcoreMesh(core_axis_name="c", subcore_axis_name="s"),
        scratch_types=[
            pltpu.VMEM((RPW, LN), jnp.float32),
            pltpu.VMEM((RPW, LN), jnp.float32),
            pltpu.VMEM((LN,), jnp.float32),
            pltpu.SemaphoreType.DMA((2,)),
        ],
    )


def _combine(part_ref, out_ref):
    a = jnp.sum(part_ref[...], axis=1)
    s = jnp.sum(a[:, 0:16], axis=1, keepdims=True)
    c = jnp.sum(a[:, 16:32], axis=1, keepdims=True)
    per_b = jnp.where(c > 0, s / jnp.where(c > 0, c, 1.0), 0.0)
    out_ref[...] = jnp.sum(per_b, axis=0, keepdims=True) * (1.0 / B)


def kernel(pred, target, label_weight):
    lw4 = label_weight.reshape(B, X, X, X)
    rv = _build_stage1()(pred, target, lw4)
    part = _build_sc_reduce()(rv)
    out = pl.pallas_call(
        _combine,
        out_shape=jax.ShapeDtypeStruct((1, 1), jnp.float32),
    )(part)
    return out[0, 0]


# speedup vs baseline: 1.0649x; 1.0649x over previous
import functools

import jax
import jax.numpy as jnp
from jax import lax
from jax.experimental import pallas as pl
from jax.experimental.pallas import tpu as pltpu
from jax.experimental.pallas import tpu_sc as plsc

B = 4
C = 12
X = 64
V = X * X * X
XCH = 16
GX = X // XCH
RPG = XCH * X
RW = GX * RPG
LN = 2 * X
NW = 32
RPW = RW // NW


def _tc_stage1(pred_ref, targ_ref, lw_ref, out_ref):
    p = pred_ref[0]
    t = targ_ref[0]
    m = jnp.max(p, axis=0)
    s = jnp.sum(jnp.exp(p - m[None]), axis=0)
    lse = m + jnp.log(s)
    tp = jnp.sum(t * p, axis=0)
    ts = jnp.sum(t, axis=0)
    kl = (lse - tp).reshape(RPG, X)
    tsf = ts.reshape(RPG, X)
    lwf = lw_ref[0].reshape(RPG, X)
    valid = jnp.logical_and(lwf > 0, tsf == 1.0)
    out_ref[0, :, 0:X] = jnp.where(valid, kl, 0.0)
    out_ref[0, :, X:LN] = jnp.where(valid, 1.0, 0.0)


@functools.cache
def _build_stage1():
    return pl.pallas_call(
        _tc_stage1,
        grid=(B, GX),
        in_specs=[
            pl.BlockSpec((1, C, XCH, X, X), lambda b, g: (b, 0, g, 0, 0)),
            pl.BlockSpec((1, C, XCH, X, X), lambda b, g: (b, 0, g, 0, 0)),
            pl.BlockSpec((1, XCH, X, X), lambda b, g: (b, g, 0, 0)),
        ],
        out_specs=pl.BlockSpec((1, RPG, LN), lambda b, g: (b, g, 0)),
        out_shape=jax.ShapeDtypeStruct((B, RW, LN), jnp.float32),
    )


def _sc_body(rv_hbm, out_hbm, rv_v0, rv_v1, st_v, sems):
    w = lax.axis_index("s") * 2 + lax.axis_index("c")
    row0 = w * RPW
    bufs = (rv_v0, rv_v1)

    def start_tile(b, slot):
        pltpu.async_copy(rv_hbm.at[b, pl.ds(row0, RPW), :], bufs[slot],
                         sems.at[slot])

    def wait_tile(slot):
        pltpu.make_async_copy(rv_hbm.at[0, pl.ds(0, RPW), :], bufs[slot],
                              sems.at[slot]).wait()

    start_tile(0, 0)
    start_tile(1, 1)
    for b in range(B):
        slot = b % 2
        wait_tile(slot)
        rv = bufs[slot]

        def y_body(y, c2, rv=rv):
            s_acc, c_acc = c2
            for zg in range(4):
                s_acc = s_acc + rv[y, pl.ds(zg * 16, 16)]
            for zg in range(4, 8):
                c_acc = c_acc + rv[y, pl.ds(zg * 16, 16)]
            return s_acc, c_acc

        acc = lax.fori_loop(
            0, RPW, y_body,
            (jnp.zeros((16,), jnp.float32), jnp.zeros((16,), jnp.float32)))

        if b + 2 < B:
            start_tile(b + 2, slot)
        st_v[pl.ds(0, 16)] = acc[0]
        st_v[pl.ds(16, 16)] = acc[1]
        pltpu.sync_copy(st_v, out_hbm.at[b, w, :])


@functools.cache
def _build_sc_reduce():
    return pl.kernel(
        _sc_body,
        out_type=jax.ShapeDtypeStruct((B, NW, LN), jnp.float32),
        mesh=plsc.VectorSubcoreMesh(core_axis_name="c", subcore_axis_name="s"),
        scratch_types=[
            pltpu.VMEM((RPW, LN), jnp.float32),
            pltpu.VMEM((RPW, LN), jnp.float32),
            pltpu.VMEM((LN,), jnp.float32),
            pltpu.SemaphoreType.DMA((2,)),
        ],
    )


def _combine(part_ref, out_ref):
    a = jnp.sum(part_ref[...], axis=1)
    s = jnp.sum(a[:, 0:16], axis=1, keepdims=True)
    c = jnp.sum(a[:, 16:32], axis=1, keepdims=True)
    per_b = jnp.where(c > 0, s / jnp.where(c > 0, c, 1.0), 0.0)
    out_ref[...] = jnp.sum(per_b, axis=0, keepdims=True) * (1.0 / B)


def kernel(pred, target, label_weight):
    lw4 = label_weight.reshape(B, X, X, X)
    rv = _build_stage1()(pred, target, lw4)
    part = _build_sc_reduce()(rv)
    out = pl.pallas_call(
        _combine,
        out_shape=jax.ShapeDtypeStruct((1, 1), jnp.float32),
    )(part)
    return out[0, 0]
